# baseline (speedup 1.0000x reference)
import jax
import jax.numpy as jnp
from jax import lax
from jax.experimental import pallas as pl
from jax.experimental.pallas import tpu as pltpu
from jax.experimental.pallas import tpu_sc as plsc

N = 10000
E = 320000
F = 128
HF = 64
NB = 64
RBLK = 2048
NP = 10240
NGRID = NP // RBLK
NCORES = 2
NSUB = 16
K = 80
ROWS_PER_TILE = 2 * NP // NSUB
WB = 80
NSLOT_PLAIN, LAG_PLAIN = 8, 4
NSLOT_CNT, LAG_CNT = 4, 2


def _mesh():
    return plsc.VectorSubcoreMesh(
        core_axis_name="c", subcore_axis_name="s",
        num_cores=NCORES, num_subcores=NSUB)


def _zero16():
    return jnp.zeros((16,), jnp.float32)


def _build_agg_body(with_counts, NSLOT, LAG):
    def body(*args):
        if with_counts:
            (src_hbm, dst_hbm, et_hbm, h_hbm, out_hbm, cnt_hbm,
             src_v, dst_v, et_v, gi_v, si_v, rows_v, ones_v,
             acc_sh, acc2_sh, *sems) = args
        else:
            (src_hbm, dst_hbm, et_hbm, h_hbm, out_hbm,
             src_v, dst_v, et_v, gi_v, si_v, rows_v,
             acc_sh, *sems) = args
            cnt_hbm = ones_v = acc2_sh = None
        isems = sems[0:NSLOT]
        gsems = sems[NSLOT:2 * NSLOT]
        ssems = sems[2 * NSLOT:3 * NSLOT]
        if with_counts:
            osems = sems[3 * NSLOT:4 * NSLOT]
            wsem, wsem2 = sems[4 * NSLOT], sems[4 * NSLOT + 1]
        else:
            wsem = sems[3 * NSLOT]
        c = lax.axis_index("c")
        s = lax.axis_index("s")
        row0 = s * ROWS_PER_TILE

        def zrow(i, carry):
            for j in range(HF // 16):
                rows_v[0, i, pl.ds(j * 16, 16)] = _zero16()
            return carry
        lax.fori_loop(0, WB, zrow, 0)

        def zcp(w, carry):
            pltpu.sync_copy(rows_v.at[0],
                            acc_sh.at[pl.ds(row0 + w * WB, WB), :])
            return carry
        lax.fori_loop(0, ROWS_PER_TILE // WB, zcp, 0)

        if with_counts:
            def z2row(i, carry):
                ones_v[i, :] = _zero16()
                return carry
            lax.fori_loop(0, K, z2row, 0)

            def z2cp(w, carry):
                pltpu.sync_copy(ones_v.at[pl.ds(0, WB), :],
                                acc2_sh.at[pl.ds(row0 + w * WB, WB), :])
                return carry
            lax.fori_loop(0, ROWS_PER_TILE // WB, z2cp, 0)

            e0 = jnp.where(lax.iota(jnp.int32, 16) == 0,
                           jnp.float32(1.0), jnp.float32(0.0))

            def orow(i, carry):
                ones_v[i, :] = e0
                return carry
            lax.fori_loop(0, K, orow, 0)
        plsc.subcore_barrier()

        t_edges = E // NSUB
        base0 = s * t_edges
        goff = c * NP
        nch = t_edges // K

        def issue_idx(i, slot):
            b = base0 + i * K
            pltpu.async_copy(src_hbm.at[pl.ds(b, K)], src_v.at[slot],
                             isems[slot])
            pltpu.async_copy(dst_hbm.at[pl.ds(b, K)], dst_v.at[slot],
                             isems[slot])
            pltpu.async_copy(et_hbm.at[pl.ds(b, K)], et_v.at[slot],
                             isems[slot])

        def wait_idx(slot):
            for _ in range(3):
                pltpu.make_async_copy(
                    src_hbm.at[pl.ds(0, K)], src_v.at[slot],
                    isems[slot]).wait()

        def compute_idx(slot):
            for j in range(K // 16):
                sl = pl.ds(j * 16, 16)
                gi_v[slot, sl] = src_v[slot, sl] + goff
                si_v[slot, sl] = dst_v[slot, sl] + et_v[slot, sl] * NP

        def wait_rows_bytes(slot, sem):
            pltpu.make_async_copy(
                h_hbm.at[pl.ds(0, K)], rows_v.at[slot], sem).wait()

        def issue_scatter(slot):
            pltpu.async_copy(rows_v.at[slot], acc_sh.at[si_v.at[slot]],
                             ssems[slot], add=True)
            if with_counts:
                pltpu.async_copy(ones_v, acc2_sh.at[si_v.at[slot]],
                                 osems[slot], add=True)

        def wait_scatter(slot):
            wait_rows_bytes(slot, ssems[slot])
            if with_counts:
                pltpu.make_async_copy(
                    cnt_hbm.at[0, pl.ds(0, K), :], ones_v,
                    osems[slot]).wait()

        def chunk(i, slot):
            @pl.when(i >= NSLOT)
            def _():
                wait_scatter(slot)
            wait_idx(slot)
            compute_idx(slot)

            @pl.when(i + NSLOT < nch)
            def _():
                issue_idx(i + NSLOT, slot)
            pltpu.async_copy(h_hbm.at[gi_v.at[slot]], rows_v.at[slot],
                             gsems[slot])
            q = (slot - LAG) % NSLOT

            @pl.when(i >= LAG)
            def _():
                wait_rows_bytes(q, gsems[q])
                issue_scatter(q)

        for p in range(NSLOT):
            issue_idx(p, p)

        nmain = (nch // NSLOT) * NSLOT

        def step(m, carry):
            for p in range(NSLOT):
                chunk(NSLOT * m + p, p)
            return carry
        lax.fori_loop(0, nch // NSLOT, step, 0)
        for i in range(nmain, nch):
            chunk(jnp.int32(i), i % NSLOT)
        for i in range(nch - LAG, nch):
            q = i % NSLOT
            wait_rows_bytes(q, gsems[q])
            issue_scatter(q)
        for i in range(nch - NSLOT, nch):
            wait_scatter(i % NSLOT)
        plsc.subcore_barrier()

        nwb = ROWS_PER_TILE // WB

        def wbi(w, carry):
            r = row0 + w * WB
            pltpu.async_copy(acc_sh.at[pl.ds(r, WB), :],
                             out_hbm.at[c, pl.ds(r, WB), :], wsem)
            if with_counts:
                pltpu.async_copy(acc2_sh.at[pl.ds(r, WB), :],
                                 cnt_hbm.at[c, pl.ds(r, WB), :], wsem2)
            return carry
        lax.fori_loop(0, nwb, wbi, 0)

        def wbw(w, carry):
            pltpu.make_async_copy(
                acc_sh.at[pl.ds(row0, WB), :],
                out_hbm.at[c, pl.ds(row0, WB), :], wsem).wait()
            if with_counts:
                pltpu.make_async_copy(
                    acc2_sh.at[pl.ds(row0, WB), :],
                    cnt_hbm.at[c, pl.ds(row0, WB), :], wsem2).wait()
            return carry
        lax.fori_loop(0, nwb, wbw, 0)
    return body


_agg_body = _build_agg_body(False, NSLOT_PLAIN, LAG_PLAIN)
_agg_cnt_body = _build_agg_body(True, NSLOT_CNT, LAG_CNT)


def _agg_scratch(nslot):
    return [
        pltpu.VMEM((nslot, K), jnp.int32),
        pltpu.VMEM((nslot, K), jnp.int32),
        pltpu.VMEM((nslot, K), jnp.int32),
        pltpu.VMEM((nslot, K), jnp.int32),
        pltpu.VMEM((nslot, K), jnp.int32),
        pltpu.VMEM((nslot, K, HF), jnp.float32),
    ]


def _sc_agg(src, dst, et, h_flat):
    return pl.kernel(
        _agg_body,
        out_type=jax.ShapeDtypeStruct((NCORES, 2 * NP, HF), jnp.float32),
        mesh=_mesh(),
        compiler_params=pltpu.CompilerParams(use_tc_tiling_on_sc=False),
        scratch_types=_agg_scratch(NSLOT_PLAIN) + [
            pltpu.VMEM_SHARED((2 * NP, HF), jnp.float32),
        ] + [pltpu.SemaphoreType.DMA] * (3 * NSLOT_PLAIN + 1),
    )(src, dst, et, h_flat)


def _sc_agg_cnt(src, dst, et, h_flat):
    return pl.kernel(
        _agg_cnt_body,
        out_type=(
            jax.ShapeDtypeStruct((NCORES, 2 * NP, HF), jnp.float32),
            jax.ShapeDtypeStruct((NCORES, 2 * NP, 16), jnp.float32),
        ),
        mesh=_mesh(),
        compiler_params=pltpu.CompilerParams(use_tc_tiling_on_sc=False),
        scratch_types=_agg_scratch(NSLOT_CNT) + [
            pltpu.VMEM((K, 16), jnp.float32),
            pltpu.VMEM_SHARED((2 * NP, HF), jnp.float32),
            pltpu.VMEM_SHARED((2 * NP, 16), jnp.float32),
        ] + [pltpu.SemaphoreType.DMA] * (4 * NSLOT_CNT + 2),
    )(src, dst, et, h_flat)


def _layer_compute(hlo, hhi, a00, a10, a01, a11, c0, c1, root_ref, w_ref,
                   b_ref):
    h = jnp.concatenate([hlo[...], hhi[...]], axis=1)
    acc = jnp.dot(h, root_ref[...],
                  preferred_element_type=jnp.float32) + b_ref[...]
    for r, (alo, ahi, cr) in enumerate(((a00, a10, c0), (a01, a11, c1))):
        a = jnp.concatenate([alo[0], ahi[0]], axis=1)
        inv = 1.0 / jnp.maximum(cr[0, :, 0:1], 1.0)
        acc = acc + jnp.dot(a * inv, w_ref[r],
                            preferred_element_type=jnp.float32)
    return jnp.maximum(acc, 0.0)


def _layer_tc_body(hlo, hhi, a00, a10, a01, a11, c0, c1, root_ref, w_ref,
                   b_ref, o_ref, out_s):
    hstep = pl.program_id(1)

    @pl.when(hstep == 0)
    def _():
        out_s[...] = _layer_compute(hlo, hhi, a00, a10, a01, a11, c0, c1,
                                    root_ref, w_ref, b_ref)
        o_ref[...] = out_s[:, :HF]

    @pl.when(hstep == 1)
    def _():
        o_ref[...] = out_s[:, HF:]


def _flat_views():
    return [
        pl.BlockSpec((1, RBLK, HF), lambda i, h, c=c, r=r:
                     (c, r * NGRID + i, 0))
        for r in range(2) for c in range(2)
    ]


def _tc_layer(h_flat, agg, counts, root, w, b2):
    hspec = [
        pl.BlockSpec((RBLK, HF), lambda i, h: (i, 0)),
        pl.BlockSpec((RBLK, HF), lambda i, h: (NGRID + i, 0)),
    ]
    cspec = [
        pl.BlockSpec((1, RBLK, 16), lambda i, h, r=r: (0, r * NGRID + i, 0))
        for r in range(2)
    ]
    wspec = [
        pl.BlockSpec((F, F), lambda i, h: (0, 0)),
        pl.BlockSpec((2, F, F), lambda i, h: (0, 0, 0)),
        pl.BlockSpec((1, F), lambda i, h: (0, 0)),
    ]
    return pl.pallas_call(
        _layer_tc_body,
        grid=(NGRID, 2),
        in_specs=hspec + _flat_views() + cspec + wspec,
        out_specs=pl.BlockSpec((RBLK, HF), lambda i, h: (h * NGRID + i, 0)),
        out_shape=jax.ShapeDtypeStruct((2 * NP, HF), jnp.float32),
        scratch_shapes=[pltpu.VMEM((RBLK, F), jnp.float32)],
    )(h_flat, h_flat, agg, agg, agg, agg, counts, counts, root, w, b2)


def _layer3_pool_body(hlo, hhi, a00, a10, a01, a11, c0, c1, root_ref,
                      w_ref, b_ref, b3_ref, wl_ref, bl_ref, o_ref,
                      s_acc, c_acc):
    i = pl.program_id(0)

    @pl.when(i == 0)
    def _():
        s_acc[...] = jnp.zeros_like(s_acc)
        c_acc[...] = jnp.zeros_like(c_acc)

    out = _layer_compute(hlo, hhi, a00, a10, a01, a11, c0, c1,
                         root_ref, w_ref, b_ref)
    bids = b3_ref[0]
    gids = lax.broadcasted_iota(jnp.int32, (NB, RBLK), 0)
    m = (gids == bids).astype(jnp.float32)
    s_acc[...] += jnp.dot(m, out, preferred_element_type=jnp.float32)
    c_acc[...] += jnp.sum(m, axis=1, keepdims=True)

    @pl.when(i == pl.num_programs(0) - 1)
    def _():
        g = s_acc[...] / jnp.maximum(c_acc[...], 1.0)
        o_ref[...] = jnp.dot(g, wl_ref[...],
                             preferred_element_type=jnp.float32) + bl_ref[...]


def _tc_layer3_pool(h_flat, agg, counts, root, w, b2, batch3, wl_pad,
                    bl_pad):
    hspec = [
        pl.BlockSpec((RBLK, HF), lambda i: (i, 0)),
        pl.BlockSpec((RBLK, HF), lambda i: (NGRID + i, 0)),
    ]
    aspec = [
        pl.BlockSpec((1, RBLK, HF), lambda i, c=c, r=r:
                     (c, r * NGRID + i, 0))
        for r in range(2) for c in range(2)
    ]
    cspec = [
        pl.BlockSpec((1, RBLK, 16), lambda i, r=r: (0, r * NGRID + i, 0))
        for r in range(2)
    ]
    return pl.pallas_call(
        _layer3_pool_body,
        grid=(NGRID,),
        in_specs=hspec + aspec + cspec + [
            pl.BlockSpec((F, F), lambda i: (0, 0)),
            pl.BlockSpec((2, F, F), lambda i: (0, 0, 0)),
            pl.BlockSpec((1, F), lambda i: (0, 0)),
            pl.BlockSpec((1, 1, RBLK), lambda i: (i, 0, 0)),
            pl.BlockSpec((F, F), lambda i: (0, 0)),
            pl.BlockSpec((1, F), lambda i: (0, 0)),
        ],
        out_specs=pl.BlockSpec((NB, F), lambda i: (0, 0)),
        out_shape=jax.ShapeDtypeStruct((NB, F), jnp.float32),
        scratch_shapes=[
            pltpu.VMEM((NB, F), jnp.float32),
            pltpu.VMEM((NB, F), jnp.float32),
        ],
    )(h_flat, h_flat, agg, agg, agg, agg, counts, counts,
      root, w, b2, batch3, wl_pad, bl_pad)


def kernel(x, edge_index, edge_attr, batch,
           W1, root1, b1, W2, root2, b2, W3, root3, b3, Wl, bl):
    src = edge_index[0].astype(jnp.int32)
    dst = edge_index[1].astype(jnp.int32)
    et = edge_attr.astype(jnp.int32)

    x_pad = jnp.zeros((NP, F), jnp.float32).at[:N].set(x)
    h_flat = jnp.concatenate([x_pad[:, :HF], x_pad[:, HF:]], axis=0)

    batch_p = jnp.concatenate(
        [batch.astype(jnp.int32), jnp.full((NP - N,), NB, jnp.int32)]
    ).reshape(NGRID, 1, RBLK)

    agg, counts = _sc_agg_cnt(src, dst, et, h_flat)
    h_flat = _tc_layer(h_flat, agg, counts, root1, W1, b1.reshape(1, F))

    agg = _sc_agg(src, dst, et, h_flat)
    h_flat = _tc_layer(h_flat, agg, counts, root2, W2, b2.reshape(1, F))

    agg = _sc_agg(src, dst, et, h_flat)
    wl_pad = jnp.zeros((F, F), jnp.float32).at[:, :Wl.shape[1]].set(Wl)
    bl_pad = jnp.zeros((1, F), jnp.float32).at[0, :bl.shape[0]].set(bl)
    out = _tc_layer3_pool(h_flat, agg, counts, root3, W3, b3.reshape(1, F),
                          batch_p, wl_pad, bl_pad)
    return out[:, :Wl.shape[1]]


# speedup vs baseline: 1.1382x; 1.1382x over previous
import jax
import jax.numpy as jnp
from jax import lax
from jax.experimental import pallas as pl
from jax.experimental.pallas import tpu as pltpu
from jax.experimental.pallas import tpu_sc as plsc

N = 10000
E = 320000
F = 128
HF = 64
NB = 64
RBLK = 2048
NP = 10240
NGRID = NP // RBLK
NCORES = 2
NSUB = 16
K = 80
ROWS_PER_TILE = 2 * NP // NSUB
WB = 80
NSLOT_PLAIN, LAG_PLAIN = 8, 4
NSLOT_CNT, LAG_CNT = 4, 2


def _mesh():
    return plsc.VectorSubcoreMesh(
        core_axis_name="c", subcore_axis_name="s",
        num_cores=NCORES, num_subcores=NSUB)


def _zero16():
    return jnp.zeros((16,), jnp.float32)


def _build_agg_body(with_counts, NSLOT, LAG):
    def body(*args):
        if with_counts:
            (src_hbm, dst_hbm, et_hbm, h_hbm, out_hbm, cnt_hbm,
             src_v, dst_v, et_v, gi_v, si_v, rows_v, ones_v,
             acc_sh, acc2_sh, *sems) = args
        else:
            (src_hbm, dst_hbm, et_hbm, h_hbm, out_hbm,
             src_v, dst_v, et_v, gi_v, si_v, rows_v,
             acc_sh, *sems) = args
            cnt_hbm = ones_v = acc2_sh = None
        isems = sems[0:NSLOT]
        gsems = sems[NSLOT:2 * NSLOT]
        ssems = sems[2 * NSLOT:3 * NSLOT]
        if with_counts:
            osems = sems[3 * NSLOT:4 * NSLOT]
            wsem, wsem2 = sems[4 * NSLOT], sems[4 * NSLOT + 1]
        else:
            wsem = sems[3 * NSLOT]
        c = lax.axis_index("c")
        s = lax.axis_index("s")
        row0 = s * ROWS_PER_TILE

        def zrow(i, carry):
            for j in range(HF // 16):
                rows_v[0, i, pl.ds(j * 16, 16)] = _zero16()
            return carry
        lax.fori_loop(0, WB, zrow, 0)

        def zcp(w, carry):
            pltpu.sync_copy(rows_v.at[0],
                            acc_sh.at[pl.ds(row0 + w * WB, WB), :])
            return carry
        lax.fori_loop(0, ROWS_PER_TILE // WB, zcp, 0)

        if with_counts:
            def z2row(i, carry):
                ones_v[i, :] = _zero16()
                return carry
            lax.fori_loop(0, K, z2row, 0)

            def z2cp(w, carry):
                pltpu.sync_copy(ones_v.at[pl.ds(0, WB), :],
                                acc2_sh.at[pl.ds(row0 + w * WB, WB), :])
                return carry
            lax.fori_loop(0, ROWS_PER_TILE // WB, z2cp, 0)

            e0 = jnp.where(lax.iota(jnp.int32, 16) == 0,
                           jnp.float32(1.0), jnp.float32(0.0))

            def orow(i, carry):
                ones_v[i, :] = e0
                return carry
            lax.fori_loop(0, K, orow, 0)
        plsc.subcore_barrier()

        t_edges = E // NSUB
        base0 = s * t_edges
        goff = c * NP
        nch = t_edges // K

        def issue_idx(i, slot):
            b = base0 + i * K
            pltpu.async_copy(src_hbm.at[pl.ds(b, K)], src_v.at[slot],
                             isems[slot])
            pltpu.async_copy(dst_hbm.at[pl.ds(b, K)], dst_v.at[slot],
                             isems[slot])
            pltpu.async_copy(et_hbm.at[pl.ds(b, K)], et_v.at[slot],
                             isems[slot])

        def wait_idx(slot):
            for _ in range(3):
                pltpu.make_async_copy(
                    src_hbm.at[pl.ds(0, K)], src_v.at[slot],
                    isems[slot]).wait()

        def compute_idx(slot):
            for j in range(K // 16):
                sl = pl.ds(j * 16, 16)
                gi_v[slot, sl] = src_v[slot, sl] + goff
                si_v[slot, sl] = dst_v[slot, sl] + et_v[slot, sl] * NP

        def wait_rows_bytes(slot, sem):
            pltpu.make_async_copy(
                h_hbm.at[pl.ds(0, K)], rows_v.at[slot], sem).wait()

        def issue_scatter(slot):
            pltpu.async_copy(rows_v.at[slot], acc_sh.at[si_v.at[slot]],
                             ssems[slot], add=True)
            if with_counts:
                pltpu.async_copy(ones_v, acc2_sh.at[si_v.at[slot]],
                                 osems[slot], add=True)

        def wait_scatter(slot):
            wait_rows_bytes(slot, ssems[slot])
            if with_counts:
                pltpu.make_async_copy(
                    cnt_hbm.at[0, pl.ds(0, K), :], ones_v,
                    osems[slot]).wait()

        def chunk(i, slot):
            @pl.when(i >= NSLOT)
            def _():
                wait_scatter(slot)
            wait_idx(slot)
            compute_idx(slot)

            @pl.when(i + NSLOT < nch)
            def _():
                issue_idx(i + NSLOT, slot)
            pltpu.async_copy(h_hbm.at[gi_v.at[slot]], rows_v.at[slot],
                             gsems[slot])
            q = (slot - LAG) % NSLOT

            @pl.when(i >= LAG)
            def _():
                wait_rows_bytes(q, gsems[q])
                issue_scatter(q)

        for p in range(NSLOT):
            issue_idx(p, p)

        nmain = (nch // NSLOT) * NSLOT

        def step(m, carry):
            for p in range(NSLOT):
                chunk(NSLOT * m + p, p)
            return carry
        lax.fori_loop(0, nch // NSLOT, step, 0)
        for i in range(nmain, nch):
            chunk(jnp.int32(i), i % NSLOT)
        for i in range(nch - LAG, nch):
            q = i % NSLOT
            wait_rows_bytes(q, gsems[q])
            issue_scatter(q)
        for i in range(nch - NSLOT, nch):
            wait_scatter(i % NSLOT)
        plsc.subcore_barrier()

        nwb = ROWS_PER_TILE // WB
        rel = s // (NSUB // 2)
        node0 = (s % (NSUB // 2)) * ROWS_PER_TILE
        coff = c * HF

        def wbi(w, carry):
            r = row0 + w * WB
            pltpu.async_copy(
                acc_sh.at[pl.ds(r, WB), :],
                out_hbm.at[rel, pl.ds(node0 + w * WB, WB),
                           pl.ds(coff, HF)], wsem)
            if with_counts:
                pltpu.async_copy(acc2_sh.at[pl.ds(r, WB), :],
                                 cnt_hbm.at[c, pl.ds(r, WB), :], wsem2)
            return carry
        lax.fori_loop(0, nwb, wbi, 0)

        def wbw(w, carry):
            pltpu.make_async_copy(
                acc_sh.at[pl.ds(row0, WB), :],
                out_hbm.at[0, pl.ds(0, WB), pl.ds(0, HF)], wsem).wait()
            if with_counts:
                pltpu.make_async_copy(
                    acc2_sh.at[pl.ds(row0, WB), :],
                    cnt_hbm.at[c, pl.ds(row0, WB), :], wsem2).wait()
            return carry
        lax.fori_loop(0, nwb, wbw, 0)
    return body


_agg_body = _build_agg_body(False, NSLOT_PLAIN, LAG_PLAIN)
_agg_cnt_body = _build_agg_body(True, NSLOT_CNT, LAG_CNT)


def _agg_scratch(nslot):
    return [
        pltpu.VMEM((nslot, K), jnp.int32),
        pltpu.VMEM((nslot, K), jnp.int32),
        pltpu.VMEM((nslot, K), jnp.int32),
        pltpu.VMEM((nslot, K), jnp.int32),
        pltpu.VMEM((nslot, K), jnp.int32),
        pltpu.VMEM((nslot, K, HF), jnp.float32),
    ]


def _sc_agg(src, dst, et, h_flat):
    return pl.kernel(
        _agg_body,
        out_type=jax.ShapeDtypeStruct((2, NP, F), jnp.float32),
        mesh=_mesh(),
        compiler_params=pltpu.CompilerParams(use_tc_tiling_on_sc=False),
        scratch_types=_agg_scratch(NSLOT_PLAIN) + [
            pltpu.VMEM_SHARED((2 * NP, HF), jnp.float32),
        ] + [pltpu.SemaphoreType.DMA] * (3 * NSLOT_PLAIN + 1),
    )(src, dst, et, h_flat)


def _sc_agg_cnt(src, dst, et, h_flat):
    return pl.kernel(
        _agg_cnt_body,
        out_type=(
            jax.ShapeDtypeStruct((2, NP, F), jnp.float32),
            jax.ShapeDtypeStruct((NCORES, 2 * NP, 16), jnp.float32),
        ),
        mesh=_mesh(),
        compiler_params=pltpu.CompilerParams(use_tc_tiling_on_sc=False),
        scratch_types=_agg_scratch(NSLOT_CNT) + [
            pltpu.VMEM((K, 16), jnp.float32),
            pltpu.VMEM_SHARED((2 * NP, HF), jnp.float32),
            pltpu.VMEM_SHARED((2 * NP, 16), jnp.float32),
        ] + [pltpu.SemaphoreType.DMA] * (4 * NSLOT_CNT + 2),
    )(src, dst, et, h_flat)


def _layer_compute(hlo, hhi, a0, a1, c0, c1, root_ref, w_ref, b_ref):
    h = jnp.concatenate([hlo[...], hhi[...]], axis=1)
    acc = jnp.dot(h, root_ref[...],
                  preferred_element_type=jnp.float32) + b_ref[...]
    for r, (ar, cr) in enumerate(((a0, c0), (a1, c1))):
        inv = 1.0 / jnp.maximum(cr[0, :, 0:1], 1.0)
        acc = acc + jnp.dot(ar[0] * inv, w_ref[r],
                            preferred_element_type=jnp.float32)
    return jnp.maximum(acc, 0.0)


def _layer_tc_body(hlo, hhi, a0, a1, c0, c1, root_ref, w_ref,
                   b_ref, o_ref, out_s):
    hstep = pl.program_id(1)

    @pl.when(hstep == 0)
    def _():
        out_s[...] = _layer_compute(hlo, hhi, a0, a1, c0, c1,
                                    root_ref, w_ref, b_ref)
        o_ref[...] = out_s[:, :HF]

    @pl.when(hstep == 1)
    def _():
        o_ref[...] = out_s[:, HF:]


def _tc_layer(h_flat, agg, counts, root, w, b2):
    hspec = [
        pl.BlockSpec((RBLK, HF), lambda i, h: (i, 0)),
        pl.BlockSpec((RBLK, HF), lambda i, h: (NGRID + i, 0)),
    ]
    aspec = [
        pl.BlockSpec((1, RBLK, F), lambda i, h, r=r: (r, i, 0))
        for r in range(2)
    ]
    cspec = [
        pl.BlockSpec((1, RBLK, 16), lambda i, h, r=r: (0, r * NGRID + i, 0))
        for r in range(2)
    ]
    wspec = [
        pl.BlockSpec((F, F), lambda i, h: (0, 0)),
        pl.BlockSpec((2, F, F), lambda i, h: (0, 0, 0)),
        pl.BlockSpec((1, F), lambda i, h: (0, 0)),
    ]
    return pl.pallas_call(
        _layer_tc_body,
        grid=(NGRID, 2),
        in_specs=hspec + aspec + cspec + wspec,
        out_specs=pl.BlockSpec((RBLK, HF), lambda i, h: (h * NGRID + i, 0)),
        out_shape=jax.ShapeDtypeStruct((2 * NP, HF), jnp.float32),
        scratch_shapes=[pltpu.VMEM((RBLK, F), jnp.float32)],
    )(h_flat, h_flat, agg, agg, counts, counts, root, w, b2)


def _layer3_pool_body(hlo, hhi, a0, a1, c0, c1, root_ref,
                      w_ref, b_ref, b3_ref, wl_ref, bl_ref, o_ref,
                      s_acc, c_acc):
    i = pl.program_id(0)

    @pl.when(i == 0)
    def _():
        s_acc[...] = jnp.zeros_like(s_acc)
        c_acc[...] = jnp.zeros_like(c_acc)

    out = _layer_compute(hlo, hhi, a0, a1, c0, c1,
                         root_ref, w_ref, b_ref)
    bids = b3_ref[0]
    gids = lax.broadcasted_iota(jnp.int32, (NB, RBLK), 0)
    m = (gids == bids).astype(jnp.float32)
    s_acc[...] += jnp.dot(m, out, preferred_element_type=jnp.float32)
    c_acc[...] += jnp.sum(m, axis=1, keepdims=True)

    @pl.when(i == pl.num_programs(0) - 1)
    def _():
        g = s_acc[...] / jnp.maximum(c_acc[...], 1.0)
        o_ref[...] = jnp.dot(g, wl_ref[...],
                             preferred_element_type=jnp.float32) + bl_ref[...]


def _tc_layer3_pool(h_flat, agg, counts, root, w, b2, batch3, wl_pad,
                    bl_pad):
    hspec = [
        pl.BlockSpec((RBLK, HF), lambda i: (i, 0)),
        pl.BlockSpec((RBLK, HF), lambda i: (NGRID + i, 0)),
    ]
    aspec = [
        pl.BlockSpec((1, RBLK, F), lambda i, r=r: (r, i, 0))
        for r in range(2)
    ]
    cspec = [
        pl.BlockSpec((1, RBLK, 16), lambda i, r=r: (0, r * NGRID + i, 0))
        for r in range(2)
    ]
    return pl.pallas_call(
        _layer3_pool_body,
        grid=(NGRID,),
        in_specs=hspec + aspec + cspec + [
            pl.BlockSpec((F, F), lambda i: (0, 0)),
            pl.BlockSpec((2, F, F), lambda i: (0, 0, 0)),
            pl.BlockSpec((1, F), lambda i: (0, 0)),
            pl.BlockSpec((1, 1, RBLK), lambda i: (i, 0, 0)),
            pl.BlockSpec((F, F), lambda i: (0, 0)),
            pl.BlockSpec((1, F), lambda i: (0, 0)),
        ],
        out_specs=pl.BlockSpec((NB, F), lambda i: (0, 0)),
        out_shape=jax.ShapeDtypeStruct((NB, F), jnp.float32),
        scratch_shapes=[
            pltpu.VMEM((NB, F), jnp.float32),
            pltpu.VMEM((NB, F), jnp.float32),
        ],
    )(h_flat, h_flat, agg, agg, counts, counts,
      root, w, b2, batch3, wl_pad, bl_pad)


def kernel(x, edge_index, edge_attr, batch,
           W1, root1, b1, W2, root2, b2, W3, root3, b3, Wl, bl):
    src = edge_index[0].astype(jnp.int32)
    dst = edge_index[1].astype(jnp.int32)
    et = edge_attr.astype(jnp.int32)

    x_pad = jnp.zeros((NP, F), jnp.float32).at[:N].set(x)
    h_flat = jnp.concatenate([x_pad[:, :HF], x_pad[:, HF:]], axis=0)

    batch_p = jnp.concatenate(
        [batch.astype(jnp.int32), jnp.full((NP - N,), NB, jnp.int32)]
    ).reshape(NGRID, 1, RBLK)

    agg, counts = _sc_agg_cnt(src, dst, et, h_flat)
    h_flat = _tc_layer(h_flat, agg, counts, root1, W1, b1.reshape(1, F))

    agg = _sc_agg(src, dst, et, h_flat)
    h_flat = _tc_layer(h_flat, agg, counts, root2, W2, b2.reshape(1, F))

    agg = _sc_agg(src, dst, et, h_flat)
    wl_pad = jnp.zeros((F, F), jnp.float32).at[:, :Wl.shape[1]].set(Wl)
    bl_pad = jnp.zeros((1, F), jnp.float32).at[0, :bl.shape[0]].set(bl)
    out = _tc_layer3_pool(h_flat, agg, counts, root3, W3, b3.reshape(1, F),
                          batch_p, wl_pad, bl_pad)
    return out[:, :Wl.shape[1]]
